# baseline (speedup 1.0000x reference)
import jax
import jax.numpy as jnp
from jax.experimental import pallas as pl
from jax.experimental.pallas import tpu as pltpu

_NORM_FACTOR = 100.0
_EPS = 1e-7
_MAXNORM = 1.0 - 1e-5


def _artanh(x):
    x = jnp.clip(x, -1.0 + _EPS, 1.0 - _EPS)
    return 0.5 * jnp.log((1.0 + x) / (1.0 - x))


def _colnorm(xT):
    return jnp.maximum(jnp.sqrt(jnp.sum(xT * xT, axis=0, keepdims=True)), 1e-15)


def _log_scale(n):
    pn = jnp.minimum(n, _MAXNORM)
    return _artanh(pn) / n


def _exp_log_scale(n):
    t = jnp.minimum(jnp.tanh(n), _MAXNORM)
    return _artanh(t) / n


def _hgcn_body(h_ref, adj_ref, maskT_ref, w1T_ref, b1_ref, w2T_ref, b2_ref,
               woT_ref, bo_ref, out_ref, adj_bf_ref):
    adj_bf_ref[...] = adj_ref[0].astype(jnp.bfloat16)

    def layer(xtT, wT_ref, bT_ref):
        msgT = jnp.dot(wT_ref[...], xtT, preferred_element_type=jnp.float32)
        msgT = msgT + bT_ref[...]
        aggT = jax.lax.dot_general(
            msgT.astype(jnp.bfloat16), adj_bf_ref[...],
            dimension_numbers=(((1,), (1,)), ((), ())),
            preferred_element_type=jnp.float32) * (1.0 / _NORM_FACTOR)
        uT = jax.nn.relu(aggT)
        return uT * _exp_log_scale(_colnorm(uT))

    hT = h_ref[0].T
    xtT = hT * _log_scale(_colnorm(hT))
    xtT = layer(xtT, w1T_ref, b1_ref)
    xtT = layer(xtT, w2T_ref, b2_ref)
    tpT = jnp.dot(woT_ref[...], xtT, preferred_element_type=jnp.float32)
    tpT = (tpT + bo_ref[...]) * maskT_ref[0]
    out_ref[0] = tpT.T


def kernel(h, adj, node_mask, W1, b1, W2, b2, W_out, b_out):
    B, N, D = h.shape
    F = W_out.shape[1]
    maskT = node_mask.reshape(B, 1, N)

    grid = (B,)
    in_specs = [
        pl.BlockSpec((1, N, D), lambda b: (b, 0, 0)),
        pl.BlockSpec((1, N, N), lambda b: (b, 0, 0)),
        pl.BlockSpec((1, 1, N), lambda b: (b, 0, 0)),
        pl.BlockSpec((D, D), lambda b: (0, 0)),
        pl.BlockSpec((D, 1), lambda b: (0, 0)),
        pl.BlockSpec((D, D), lambda b: (0, 0)),
        pl.BlockSpec((D, 1), lambda b: (0, 0)),
        pl.BlockSpec((F, D), lambda b: (0, 0)),
        pl.BlockSpec((F, 1), lambda b: (0, 0)),
    ]
    out_spec = pl.BlockSpec((1, N, F), lambda b: (b, 0, 0))

    return pl.pallas_call(
        _hgcn_body,
        grid=grid,
        in_specs=in_specs,
        out_specs=out_spec,
        out_shape=jax.ShapeDtypeStruct((B, N, F), jnp.float32),
        scratch_shapes=[pltpu.VMEM((N, N), jnp.bfloat16)],
    )(h, adj, maskT, W1.T, b1.reshape(D, 1), W2.T, b2.reshape(D, 1),
      W_out.T, b_out.reshape(F, 1))


# speedup vs baseline: 1.1118x; 1.1118x over previous
import jax
import jax.numpy as jnp
from jax.experimental import pallas as pl
from jax.experimental.pallas import tpu as pltpu

_NORM_FACTOR = 100.0
_EPS = 1e-7
_MAXNORM = 1.0 - 1e-5


def _artanh(x):
    x = jnp.clip(x, -1.0 + _EPS, 1.0 - _EPS)
    return 0.5 * jnp.log((1.0 + x) / (1.0 - x))


def _colnorm(xT):
    return jnp.maximum(jnp.sqrt(jnp.sum(xT * xT, axis=0, keepdims=True)), 1e-15)


def _log_scale(n):
    pn = jnp.minimum(n, _MAXNORM)
    return _artanh(pn) / n


def _exp_log_scale(n):
    t = jnp.minimum(jnp.tanh(n), _MAXNORM)
    return _artanh(t) / n


def _hgcn_body(hT_ref, adj_ref, maskT_ref, w1T_ref, b1_ref, w2T_ref, b2_ref,
               woT_ref, bo_ref, out_ref, adj_bf_ref):
    adj_bf_ref[...] = adj_ref[0].astype(jnp.bfloat16)

    def layer(xtT, wT_ref, bT_ref):
        msgT = jnp.dot(wT_ref[...], xtT, preferred_element_type=jnp.float32)
        msgT = msgT + bT_ref[...]
        aggT = jax.lax.dot_general(
            msgT.astype(jnp.bfloat16), adj_bf_ref[...],
            dimension_numbers=(((1,), (1,)), ((), ())),
            preferred_element_type=jnp.float32) * (1.0 / _NORM_FACTOR)
        uT = jax.nn.relu(aggT)
        return uT * _exp_log_scale(_colnorm(uT))

    hT = hT_ref[0]
    xtT = hT * _log_scale(_colnorm(hT))
    xtT = layer(xtT, w1T_ref, b1_ref)
    xtT = layer(xtT, w2T_ref, b2_ref)
    tpT = jnp.dot(woT_ref[...], xtT, preferred_element_type=jnp.float32)
    tpT = (tpT + bo_ref[...]) * maskT_ref[0]
    out_ref[0] = tpT.T


def kernel(h, adj, node_mask, W1, b1, W2, b2, W_out, b_out):
    B, N, D = h.shape
    F = W_out.shape[1]
    hT = jnp.swapaxes(h, 1, 2)
    maskT = jnp.swapaxes(node_mask, 1, 2)
    b1_2d = b1.reshape(D, 1)
    b2_2d = b2.reshape(D, 1)
    bo_2d = b_out.reshape(F, 1)

    grid = (B,)
    in_specs = [
        pl.BlockSpec((1, D, N), lambda b: (b, 0, 0)),
        pl.BlockSpec((1, N, N), lambda b: (b, 0, 0)),
        pl.BlockSpec((1, 1, N), lambda b: (b, 0, 0)),
        pl.BlockSpec((D, D), lambda b: (0, 0)),
        pl.BlockSpec((D, 1), lambda b: (0, 0)),
        pl.BlockSpec((D, D), lambda b: (0, 0)),
        pl.BlockSpec((D, 1), lambda b: (0, 0)),
        pl.BlockSpec((F, D), lambda b: (0, 0)),
        pl.BlockSpec((F, 1), lambda b: (0, 0)),
    ]
    out_spec = pl.BlockSpec((1, N, F), lambda b: (b, 0, 0))

    return pl.pallas_call(
        _hgcn_body,
        grid=grid,
        in_specs=in_specs,
        out_specs=out_spec,
        out_shape=jax.ShapeDtypeStruct((B, N, F), jnp.float32),
        scratch_shapes=[pltpu.VMEM((N, N), jnp.bfloat16)],
    )(hT, adj, maskT, W1.T, b1_2d, W2.T, b2_2d, W_out.T, bo_2d)
